# Initial kernel scaffold; baseline (speedup 1.0000x reference)
import jax
import jax.numpy as jnp
from jax.experimental import pallas as pl


def kernel(p):
    raise NotImplementedError("write your pallas kernel here")


# speedup vs baseline: 2.1056x; 2.1056x over previous
import functools

import jax
import jax.numpy as jnp
from jax.experimental import pallas as pl
from jax.experimental.pallas import tpu as pltpu

_K = 32
_BR = 256


def _topk_body(xr_ref, yr_ref, sqr_ref, xc_ref, yc_ref, sqc_ref, out_ref):
    i = pl.program_id(0)
    br = xr_ref.shape[0]
    n = xc_ref.shape[1]
    xr = xr_ref[...]
    yr = yr_ref[...]
    sqr = sqr_ref[...]
    xc = xc_ref[...]
    yc = yc_ref[...]
    sqc = sqc_ref[...]

    cross = xr * xc + yr * yc
    d2 = (sqr + sqc) - 2.0 * cross
    neg = -d2

    colid = jax.lax.broadcasted_iota(jnp.int32, (1, n), 1)
    row_ids = i * br + jax.lax.broadcasted_iota(jnp.int32, (br, 1), 0)
    neg = jnp.where(colid == row_ids, -jnp.inf, neg)

    sels = []
    for _ in range(_K):
        g = jnp.max(neg, axis=1, keepdims=True)
        cand = jnp.where(neg == g, colid, n)
        sel = jnp.min(cand, axis=1, keepdims=True)
        sels.append(sel)
        neg = jnp.where(colid == sel, -jnp.inf, neg)
    out_ref[...] = jnp.concatenate(sels, axis=1)


def _topk_indices(p):
    n = p.shape[0]
    x = p[:, 0].astype(jnp.bfloat16).astype(jnp.float32)
    y = p[:, 1].astype(jnp.bfloat16).astype(jnp.float32)
    sq = p[:, 0] * p[:, 0] + p[:, 1] * p[:, 1]
    xr = x.reshape(n, 1)
    yr = y.reshape(n, 1)
    sqr = sq.reshape(n, 1)
    xc = x.reshape(1, n)
    yc = y.reshape(1, n)
    sqc = sq.reshape(1, n)
    grid = n // _BR
    row_spec = pl.BlockSpec((_BR, 1), lambda i: (i, 0))
    col_spec = pl.BlockSpec((1, n), lambda i: (0, 0))
    return pl.pallas_call(
        _topk_body,
        grid=(grid,),
        in_specs=[row_spec, row_spec, row_spec, col_spec, col_spec, col_spec],
        out_specs=pl.BlockSpec((_BR, _K), lambda i: (i, 0)),
        out_shape=jax.ShapeDtypeStruct((n, _K), jnp.int32),
        compiler_params=pltpu.CompilerParams(
            dimension_semantics=("parallel",),
        ),
    )(xr, yr, sqr, xc, yc, sqc)


def kernel(p):
    idx = _topk_indices(p)
    disp = p[:, None, :2] - p[idx, :2]
    return disp


# speedup vs baseline: 4.3639x; 2.0725x over previous
import functools

import jax
import jax.numpy as jnp
from jax import lax
from jax.experimental import pallas as pl
from jax.experimental.pallas import tpu as pltpu
from jax.experimental.pallas import tpu_sc as plsc

_K = 32
_BR = 256


def _topk_body(xr_ref, yr_ref, sqr_ref, xc_ref, yc_ref, sqc_ref, out_ref):
    i = pl.program_id(0)
    br = xr_ref.shape[0]
    n = xc_ref.shape[1]
    xr = xr_ref[...]
    yr = yr_ref[...]
    sqr = sqr_ref[...]
    xc = xc_ref[...]
    yc = yc_ref[...]
    sqc = sqc_ref[...]

    cross = xr * xc + yr * yc
    d2 = (sqr + sqc) - 2.0 * cross
    neg = -d2

    colid = jax.lax.broadcasted_iota(jnp.int32, (1, n), 1)
    row_ids = i * br + jax.lax.broadcasted_iota(jnp.int32, (br, 1), 0)
    neg = jnp.where(colid == row_ids, -jnp.inf, neg)

    sels = []
    for _ in range(_K):
        g = jnp.max(neg, axis=1, keepdims=True)
        cand = jnp.where(neg == g, colid, n)
        sel = jnp.min(cand, axis=1, keepdims=True)
        sels.append(sel)
        neg = jnp.where(colid == sel, -jnp.inf, neg)
    out_ref[...] = jnp.concatenate(sels, axis=1)


def _topk_indices(p):
    n = p.shape[0]
    x = p[:, 0].astype(jnp.bfloat16).astype(jnp.float32)
    y = p[:, 1].astype(jnp.bfloat16).astype(jnp.float32)
    sq = p[:, 0] * p[:, 0] + p[:, 1] * p[:, 1]
    xr = x.reshape(n, 1)
    yr = y.reshape(n, 1)
    sqr = sq.reshape(n, 1)
    xc = x.reshape(1, n)
    yc = y.reshape(1, n)
    sqc = sq.reshape(1, n)
    grid = n // _BR
    row_spec = pl.BlockSpec((_BR, 1), lambda i: (i, 0))
    col_spec = pl.BlockSpec((1, n), lambda i: (0, 0))
    return pl.pallas_call(
        _topk_body,
        grid=(grid,),
        in_specs=[row_spec, row_spec, row_spec, col_spec, col_spec, col_spec],
        out_specs=pl.BlockSpec((_BR, _K), lambda i: (i, 0)),
        out_shape=jax.ShapeDtypeStruct((n, _K), jnp.int32),
        compiler_params=pltpu.CompilerParams(
            dimension_semantics=("parallel",),
        ),
    )(xr, yr, sqr, xc, yc, sqc)


def _disp_gather(p, idx):
    n, k = idx.shape
    info = plsc.get_sparse_core_info()
    nc = info.num_cores
    nw = nc * info.num_subcores
    rw = n // nw
    x = p[:, 0]
    y = p[:, 1]
    mesh = plsc.VectorSubcoreMesh(core_axis_name="c", subcore_axis_name="s")

    @functools.partial(
        pl.kernel,
        mesh=mesh,
        compiler_params=pltpu.CompilerParams(
            use_tc_tiling_on_sc=False, needs_layout_passes=False
        ),
        out_type=jax.ShapeDtypeStruct((n * k * 2,), jnp.float32),
        scratch_types=[
            pltpu.VMEM((n,), jnp.float32),
            pltpu.VMEM((n,), jnp.float32),
            pltpu.VMEM((rw, k), jnp.int32),
            pltpu.VMEM((rw * k * 2,), jnp.float32),
        ],
    )
    def sc_gather(x_hbm, y_hbm, idx_hbm, out_hbm, xv, yv, idxv, outv):
        wid = lax.axis_index("s") * nc + lax.axis_index("c")
        base = wid * rw
        pltpu.sync_copy(x_hbm, xv)
        pltpu.sync_copy(y_hbm, yv)
        pltpu.sync_copy(idx_hbm.at[pl.ds(base, rw)], idxv)
        iota16 = lax.iota(jnp.int32, 16)

        def row_body(r, carry):
            rvec = jnp.full((16,), base + r, jnp.int32)
            rx = plsc.load_gather(xv, [rvec])
            ry = plsc.load_gather(yv, [rvec])
            for v in range(k // 16):
                iv = idxv[r, pl.ds(v * 16, 16)]
                gx = plsc.load_gather(xv, [iv])
                gy = plsc.load_gather(yv, [iv])
                pos = r * (k * 2) + v * 32 + 2 * iota16
                plsc.store_scatter(outv, [pos], rx - gx)
                plsc.store_scatter(outv, [pos + 1], ry - gy)
            return carry

        lax.fori_loop(0, rw, row_body, 0)
        pltpu.sync_copy(outv, out_hbm.at[pl.ds(base * k * 2, rw * k * 2)])

    return sc_gather(x, y, idx).reshape(n, k, 2)


def kernel(p):
    idx = _topk_indices(p)
    return _disp_gather(p, idx)
